# Initial kernel scaffold; baseline (speedup 1.0000x reference)
import jax
import jax.numpy as jnp
from jax.experimental import pallas as pl


def kernel(features, edge_index, W0, b0, gamma0, beta0, W1, b1, gamma1, beta1, W2, b2, gamma2, beta2):
    raise NotImplementedError("write your pallas kernel here")


# speedup vs baseline: 4.3556x; 4.3556x over previous
import functools

import jax
import jax.numpy as jnp
from jax import lax
from jax.experimental import pallas as pl
from jax.experimental.pallas import tpu as pltpu
from jax.experimental.pallas import tpu_sc as plsc

_N = 10000
_D = 128
_E = 320000
_NC = 2
_NS = 16
_NW = _NC * _NS
_RPW = 320
_NPAD = _NW * _RPW
_PK = 16384
_CHUNK = 8000
_NCHUNKS = _E // _CHUNK
_FLUSH = 8192
_STG = _FLUSH + _CHUNK + 192
_CAP = _E + _STG
_G = 256

_mesh = plsc.VectorSubcoreMesh(core_axis_name="c", subcore_axis_name="s")


def _wid():
    return lax.axis_index("s") * _NC + lax.axis_index("c")


@functools.partial(
    pl.kernel,
    out_type=[
        jax.ShapeDtypeStruct((_NW * _CAP,), jnp.int32),
        jax.ShapeDtypeStruct((_NW * 16,), jnp.int32),
    ],
    mesh=_mesh,
    compiler_params=pltpu.CompilerParams(needs_layout_passes=False),
    scratch_types=[
        pltpu.VMEM((_CHUNK,), jnp.int32),
        pltpu.VMEM((_CHUNK,), jnp.int32),
        pltpu.VMEM((_STG,), jnp.int32),
        pltpu.VMEM((16,), jnp.int32),
        pltpu.SemaphoreType.DMA,
        pltpu.SemaphoreType.DMA,
    ],
)
def _bin_edges(pk_hbm, binb_hbm, cnt_hbm, c0, c1, stg, cvec, sem0, sem1):
    w = _wid()
    lo = w * _RPW
    hi = lo + _RPW

    def process(buf, carry):
        def group_body(i, off):
            pv = buf[pl.ds(i * 16, 16)]
            q = pv // _PK
            vd = pv - q * _PK
            m = (vd >= lo) & (vd < hi)
            csum = plsc.cumsum(m.astype(jnp.int32))
            plsc.store_scatter(stg, [csum + (off - 1)], pv, mask=m)
            return off + jnp.max(csum)

        off, written = carry
        off = lax.fori_loop(0, _CHUNK // 16, group_body, off)

        def do_flush(off, written):
            pltpu.sync_copy(
                stg.at[pl.ds(0, _FLUSH)],
                binb_hbm.at[pl.ds(pl.multiple_of(w * _CAP + written, 8), _FLUSH)])
            def shift_body(i, _):
                stg[pl.ds(i * 16, 16)] = stg[pl.ds(_FLUSH + i * 16, 16)]
                return 0
            lax.fori_loop(0, (_STG - _FLUSH) // 16, shift_body, 0)
            return off - _FLUSH, written + _FLUSH

        def no_flush(off, written):
            return off, written

        return lax.cond(off >= _FLUSH, do_flush, no_flush, off, written)

    pltpu.async_copy(pk_hbm.at[pl.ds(0, _CHUNK)], c0, sem0)

    def pair(p, carry):
        c = 2 * p
        pltpu.make_async_copy(pk_hbm.at[pl.ds(c * _CHUNK, _CHUNK)], c0, sem0).wait()
        pltpu.async_copy(pk_hbm.at[pl.ds((c + 1) * _CHUNK, _CHUNK)], c1, sem1)
        carry = process(c0, carry)
        pltpu.make_async_copy(pk_hbm.at[pl.ds((c + 1) * _CHUNK, _CHUNK)], c1, sem1).wait()

        @pl.when(c + 2 < _NCHUNKS)
        def _():
            pltpu.async_copy(pk_hbm.at[pl.ds((c + 2) * _CHUNK, _CHUNK)], c0, sem0)

        return process(c1, carry)

    off, written = lax.fori_loop(0, _NCHUNKS // 2, pair, (0, 0))
    pltpu.sync_copy(stg, binb_hbm.at[pl.ds(pl.multiple_of(w * _CAP + written, 8), _STG)])
    cvec[...] = jnp.full((16,), written + off, jnp.int32)
    pltpu.sync_copy(cvec, cnt_hbm.at[pl.ds(pl.multiple_of(w * 16, 8), 16)])


@functools.partial(
    pl.kernel,
    out_type=jax.ShapeDtypeStruct((_NPAD, _D), jnp.float32),
    mesh=_mesh,
    compiler_params=pltpu.CompilerParams(needs_layout_passes=False),
    scratch_types=[
        pltpu.VMEM((_RPW, _D), jnp.float32),
        pltpu.VMEM((_G,), jnp.int32),
        pltpu.VMEM((_G,), jnp.int32),
        pltpu.VMEM((_G,), jnp.int32),
        pltpu.VMEM((_G,), jnp.int32),
        pltpu.VMEM((_G + 16,), jnp.int32),
        pltpu.VMEM((_G + 16,), jnp.int32),
        pltpu.VMEM((_G, _D), jnp.float32),
        pltpu.VMEM((_G, _D), jnp.float32),
        pltpu.VMEM((16,), jnp.int32),
        pltpu.SemaphoreType.DMA,
        pltpu.SemaphoreType.DMA,
    ],
)
def _gather_max(h_hbm, binb_hbm, cnt_hbm, m_hbm,
                acc, pk0, pk1, sidx0, sidx1, dl0, dl1, rows0, rows1,
                cvec, sem0, sem1):
    w = _wid()
    lo = w * _RPW
    pltpu.sync_copy(h_hbm.at[pl.ds(pl.multiple_of(lo, 8), _RPW)], acc)
    pltpu.sync_copy(cnt_hbm.at[pl.ds(pl.multiple_of(w * 16, 8), 16)], cvec)
    cnt = jnp.max(cvec[...])
    ngroups = (cnt + _G - 1) // _G

    def fetch(g, pk, sidx, dl, rows, sem):
        base = g * _G
        pltpu.sync_copy(binb_hbm.at[pl.ds(pl.multiple_of(w * _CAP + base, 8), _G)], pk)

        def unpack(i, _):
            pv = pk[pl.ds(i * 16, 16)]
            q = pv // _PK
            sidx[pl.ds(i * 16, 16)] = jnp.minimum(jnp.maximum(q, 0), _NPAD - 1)
            dl[pl.ds(i * 16, 16)] = pv - q * _PK - lo
            return 0

        lax.fori_loop(0, _G // 16, unpack, 0)
        pltpu.async_copy(h_hbm.at[sidx], rows, sem)

    def process(g, dl, rows):
        ne = jnp.maximum(0, jnp.minimum(_G, cnt - g * _G))

        def quad_body(p, _):
            e = p * 4
            for k in range(4):
                d = dl[pl.ds(e + k, 16)][0]
                for b in range(_D // 16):
                    sl = pl.ds(b * 16, 16)
                    acc[d, sl] = jnp.maximum(acc[d, sl], rows[e + k, sl])
            return 0

        lax.fori_loop(0, ne // 4, quad_body, 0)

        def tail_body(t, _):
            e = (ne // 4) * 4 + t
            d = dl[pl.ds(e, 16)][0]
            for b in range(_D // 16):
                sl = pl.ds(b * 16, 16)
                acc[d, sl] = jnp.maximum(acc[d, sl], rows[e, sl])
            return 0

        lax.fori_loop(0, ne % 4, tail_body, 0)

    @pl.when(ngroups > 0)
    def _():
        fetch(0, pk0, sidx0, dl0, rows0, sem0)

    def pair(p, _):
        g0 = 2 * p
        g1 = g0 + 1
        pltpu.make_async_copy(h_hbm.at[sidx0], rows0, sem0).wait()

        @pl.when(g1 < ngroups)
        def _():
            fetch(g1, pk1, sidx1, dl1, rows1, sem1)

        process(g0, dl0, rows0)

        @pl.when(g1 < ngroups)
        def _():
            pltpu.make_async_copy(h_hbm.at[sidx1], rows1, sem1).wait()

            @pl.when(g1 + 1 < ngroups)
            def _():
                fetch(g1 + 1, pk0, sidx0, dl0, rows0, sem0)

            process(g1, dl1, rows1)

        return 0

    lax.fori_loop(0, (ngroups + 1) // 2, pair, 0)
    pltpu.sync_copy(acc, m_hbm.at[pl.ds(pl.multiple_of(lo, 8), _RPW)])


def _dense_body(h_ref, m_ref, w_ref, b_ref, g_ref, be_ref, o_ref):
    hh = h_ref[...]
    rel = m_ref[...] - hh
    z = jnp.dot(hh, w_ref[:_D, :], preferred_element_type=jnp.float32)
    z = z + jnp.dot(rel, w_ref[_D:, :], preferred_element_type=jnp.float32)
    z = z + b_ref[...]
    mu = jnp.mean(z, axis=-1, keepdims=True)
    var = jnp.mean((z - mu) ** 2, axis=-1, keepdims=True)
    y = (z - mu) * lax.rsqrt(var + 1e-5) * g_ref[...] + be_ref[...]
    o_ref[...] = jnp.where(y > 0, y, jnp.exp(jnp.minimum(y, 0.0)) - 1.0)


_BLK = 512


def _dense(h, m, W, b, gamma, beta):
    return pl.pallas_call(
        _dense_body,
        grid=(_NPAD // _BLK,),
        in_specs=[
            pl.BlockSpec((_BLK, _D), lambda i: (i, 0)),
            pl.BlockSpec((_BLK, _D), lambda i: (i, 0)),
            pl.BlockSpec((2 * _D, _D), lambda i: (0, 0)),
            pl.BlockSpec((1, _D), lambda i: (0, 0)),
            pl.BlockSpec((1, _D), lambda i: (0, 0)),
            pl.BlockSpec((1, _D), lambda i: (0, 0)),
        ],
        out_specs=pl.BlockSpec((_BLK, _D), lambda i: (i, 0)),
        out_shape=jax.ShapeDtypeStruct((_NPAD, _D), jnp.float32),
    )(h, m, W, b[None], gamma[None], beta[None])


def kernel(features, edge_index, W0, b0, gamma0, beta0,
           W1, b1, gamma1, beta1, W2, b2, gamma2, beta2):
    src = edge_index[0].astype(jnp.int32)
    dst = edge_index[1].astype(jnp.int32)
    pk = src * _PK + dst
    h = jnp.pad(features, ((0, _NPAD - _N), (0, 0)))
    binb, cnts = _bin_edges(pk)
    for (W, b, g, be) in ((W0, b0, gamma0, beta0),
                          (W1, b1, gamma1, beta1),
                          (W2, b2, gamma2, beta2)):
        m = _gather_max(h, binb, cnts)
        h = _dense(h, m, W, b, g, be)
    return h[:_N]


# speedup vs baseline: 5.1157x; 1.1745x over previous
import functools

import jax
import jax.numpy as jnp
from jax import lax
from jax.experimental import pallas as pl
from jax.experimental.pallas import tpu as pltpu
from jax.experimental.pallas import tpu_sc as plsc

_N = 10000
_D = 128
_E = 320000
_NC = 2
_NS = 16
_NW = _NC * _NS
_RPW = 320
_NPAD = _NW * _RPW
_PK = 16384
_CHUNK = 8000
_NCHUNKS = _E // _CHUNK
_FLUSH = 8192
_STG = _FLUSH + _CHUNK + 192
_CAP = _E + _STG
_G = 256

_mesh = plsc.VectorSubcoreMesh(core_axis_name="c", subcore_axis_name="s")


def _wid():
    return lax.axis_index("s") * _NC + lax.axis_index("c")


@functools.partial(
    pl.kernel,
    out_type=[
        jax.ShapeDtypeStruct((_NW * _CAP,), jnp.int32),
        jax.ShapeDtypeStruct((_NW * 16,), jnp.int32),
    ],
    mesh=_mesh,
    compiler_params=pltpu.CompilerParams(needs_layout_passes=False),
    scratch_types=[
        pltpu.VMEM((_CHUNK,), jnp.int32),
        pltpu.VMEM((_CHUNK,), jnp.int32),
        pltpu.VMEM((_STG,), jnp.int32),
        pltpu.VMEM((16,), jnp.int32),
        pltpu.SemaphoreType.DMA,
        pltpu.SemaphoreType.DMA,
    ],
)
def _bin_edges(pk_hbm, binb_hbm, cnt_hbm, c0, c1, stg, cvec, sem0, sem1):
    w = _wid()
    lo = w * _RPW
    hi = lo + _RPW

    def process(buf, carry):
        def group_body(i, off):
            pv = buf[pl.ds(i * 16, 16)]
            q = pv // _PK
            vd = pv - q * _PK
            m = (vd >= lo) & (vd < hi)
            csum = plsc.cumsum(m.astype(jnp.int32))
            plsc.store_scatter(stg, [csum + (off - 1)], pv, mask=m)
            return off + jnp.max(csum)

        off, written = carry
        off = lax.fori_loop(0, _CHUNK // 16, group_body, off)

        def do_flush(off, written):
            pltpu.sync_copy(
                stg.at[pl.ds(0, _FLUSH)],
                binb_hbm.at[pl.ds(pl.multiple_of(w * _CAP + written, 8), _FLUSH)])
            def shift_body(i, _):
                stg[pl.ds(i * 16, 16)] = stg[pl.ds(_FLUSH + i * 16, 16)]
                return 0
            lax.fori_loop(0, (_STG - _FLUSH) // 16, shift_body, 0)
            return off - _FLUSH, written + _FLUSH

        def no_flush(off, written):
            return off, written

        return lax.cond(off >= _FLUSH, do_flush, no_flush, off, written)

    pltpu.async_copy(pk_hbm.at[pl.ds(0, _CHUNK)], c0, sem0)

    def pair(p, carry):
        c = 2 * p
        pltpu.make_async_copy(pk_hbm.at[pl.ds(c * _CHUNK, _CHUNK)], c0, sem0).wait()
        pltpu.async_copy(pk_hbm.at[pl.ds((c + 1) * _CHUNK, _CHUNK)], c1, sem1)
        carry = process(c0, carry)
        pltpu.make_async_copy(pk_hbm.at[pl.ds((c + 1) * _CHUNK, _CHUNK)], c1, sem1).wait()

        @pl.when(c + 2 < _NCHUNKS)
        def _():
            pltpu.async_copy(pk_hbm.at[pl.ds((c + 2) * _CHUNK, _CHUNK)], c0, sem0)

        return process(c1, carry)

    off, written = lax.fori_loop(0, _NCHUNKS // 2, pair, (0, 0))
    pltpu.sync_copy(stg, binb_hbm.at[pl.ds(pl.multiple_of(w * _CAP + written, 8), _STG)])
    cvec[...] = jnp.full((16,), written + off, jnp.int32)
    pltpu.sync_copy(cvec, cnt_hbm.at[pl.ds(pl.multiple_of(w * 16, 8), 16)])


@functools.partial(
    pl.kernel,
    out_type=jax.ShapeDtypeStruct((_NPAD, _D), jnp.float32),
    mesh=_mesh,
    compiler_params=pltpu.CompilerParams(needs_layout_passes=False),
    scratch_types=[
        pltpu.VMEM((_RPW, _D), jnp.float32),
        pltpu.VMEM((_G,), jnp.int32),
        pltpu.VMEM((_G,), jnp.int32),
        pltpu.VMEM((_G,), jnp.int32),
        pltpu.VMEM((_G,), jnp.int32),
        pltpu.VMEM((_G + 16,), jnp.int32),
        pltpu.VMEM((_G + 16,), jnp.int32),
        pltpu.VMEM((_G, _D), jnp.float32),
        pltpu.VMEM((_G, _D), jnp.float32),
        pltpu.VMEM((16,), jnp.int32),
        pltpu.SemaphoreType.DMA,
        pltpu.SemaphoreType.DMA,
        pltpu.SemaphoreType.DMA,
        pltpu.SemaphoreType.DMA,
    ],
)
def _gather_max(h_hbm, binb_hbm, cnt_hbm, m_hbm,
                acc, pk0, pk1, sidx0, sidx1, dl0, dl1, rows0, rows1,
                cvec, sem0, sem1, psem0, psem1):
    w = _wid()
    lo = w * _RPW
    pltpu.sync_copy(h_hbm.at[pl.ds(pl.multiple_of(lo, 8), _RPW)], acc)
    pltpu.sync_copy(cnt_hbm.at[pl.ds(pl.multiple_of(w * 16, 8), 16)], cvec)
    cnt = jnp.max(cvec[...])
    ngroups = (cnt + _G - 1) // _G

    def pk_src(g):
        return binb_hbm.at[pl.ds(pl.multiple_of(w * _CAP + g * _G, 8), _G)]

    def unpack_issue(pk, sidx, dl, rows, sem):
        def unpack(i, _):
            pv = pk[pl.ds(i * 16, 16)]
            q = pv // _PK
            sidx[pl.ds(i * 16, 16)] = jnp.minimum(jnp.maximum(q, 0), _NPAD - 1)
            dl[pl.ds(i * 16, 16)] = pv - q * _PK - lo
            return 0

        lax.fori_loop(0, _G // 16, unpack, 0)
        pltpu.async_copy(h_hbm.at[sidx], rows, sem)

    def process(g, dl, rows):
        ne = jnp.maximum(0, jnp.minimum(_G, cnt - g * _G))

        def quad_body(p, _):
            e = p * 4
            v = dl[pl.ds(e, 16)]
            for k in range(4):
                d = v[k]
                for b in range(_D // 16):
                    sl = pl.ds(b * 16, 16)
                    acc[d, sl] = jnp.maximum(acc[d, sl], rows[e + k, sl])
            return 0

        lax.fori_loop(0, ne // 4, quad_body, 0)

        def tail_body(t, _):
            e = (ne // 4) * 4 + t
            d = dl[pl.ds(e, 16)][0]
            for b in range(_D // 16):
                sl = pl.ds(b * 16, 16)
                acc[d, sl] = jnp.maximum(acc[d, sl], rows[e, sl])
            return 0

        lax.fori_loop(0, ne % 4, tail_body, 0)

    @pl.when(ngroups > 0)
    def _():
        pltpu.async_copy(pk_src(0), pk0, psem0)
        pltpu.make_async_copy(pk_src(0), pk0, psem0).wait()
        unpack_issue(pk0, sidx0, dl0, rows0, sem0)

        @pl.when(1 < ngroups)
        def _():
            pltpu.async_copy(pk_src(1), pk1, psem1)

    def pair(p, _):
        g0 = 2 * p
        g1 = g0 + 1
        g2 = g0 + 2

        @pl.when(g1 < ngroups)
        def _():
            pltpu.make_async_copy(pk_src(g1), pk1, psem1).wait()

            @pl.when(g2 < ngroups)
            def _():
                pltpu.async_copy(pk_src(g2), pk0, psem0)

            unpack_issue(pk1, sidx1, dl1, rows1, sem1)

        pltpu.make_async_copy(h_hbm.at[sidx0], rows0, sem0).wait()
        process(g0, dl0, rows0)

        @pl.when(g1 < ngroups)
        def _():
            @pl.when(g2 < ngroups)
            def _():
                pltpu.make_async_copy(pk_src(g2), pk0, psem0).wait()

                @pl.when(g2 + 1 < ngroups)
                def _():
                    pltpu.async_copy(pk_src(g2 + 1), pk1, psem1)

                unpack_issue(pk0, sidx0, dl0, rows0, sem0)

            pltpu.make_async_copy(h_hbm.at[sidx1], rows1, sem1).wait()
            process(g1, dl1, rows1)

        return 0

    lax.fori_loop(0, (ngroups + 1) // 2, pair, 0)
    pltpu.sync_copy(acc, m_hbm.at[pl.ds(pl.multiple_of(lo, 8), _RPW)])


def _dense_body(h_ref, m_ref, w_ref, b_ref, g_ref, be_ref, o_ref):
    hh = h_ref[...]
    rel = m_ref[...] - hh
    z = jnp.dot(hh, w_ref[:_D, :], preferred_element_type=jnp.float32)
    z = z + jnp.dot(rel, w_ref[_D:, :], preferred_element_type=jnp.float32)
    z = z + b_ref[...]
    mu = jnp.mean(z, axis=-1, keepdims=True)
    var = jnp.mean((z - mu) ** 2, axis=-1, keepdims=True)
    y = (z - mu) * lax.rsqrt(var + 1e-5) * g_ref[...] + be_ref[...]
    o_ref[...] = jnp.where(y > 0, y, jnp.exp(jnp.minimum(y, 0.0)) - 1.0)


_BLK = 512


def _dense(h, m, W, b, gamma, beta):
    return pl.pallas_call(
        _dense_body,
        grid=(_NPAD // _BLK,),
        in_specs=[
            pl.BlockSpec((_BLK, _D), lambda i: (i, 0)),
            pl.BlockSpec((_BLK, _D), lambda i: (i, 0)),
            pl.BlockSpec((2 * _D, _D), lambda i: (0, 0)),
            pl.BlockSpec((1, _D), lambda i: (0, 0)),
            pl.BlockSpec((1, _D), lambda i: (0, 0)),
            pl.BlockSpec((1, _D), lambda i: (0, 0)),
        ],
        out_specs=pl.BlockSpec((_BLK, _D), lambda i: (i, 0)),
        out_shape=jax.ShapeDtypeStruct((_NPAD, _D), jnp.float32),
    )(h, m, W, b[None], gamma[None], beta[None])


def kernel(features, edge_index, W0, b0, gamma0, beta0,
           W1, b1, gamma1, beta1, W2, b2, gamma2, beta2):
    src = edge_index[0].astype(jnp.int32)
    dst = edge_index[1].astype(jnp.int32)
    pk = src * _PK + dst
    h = jnp.pad(features, ((0, _NPAD - _N), (0, 0)))
    binb, cnts = _bin_edges(pk)
    for (W, b, g, be) in ((W0, b0, gamma0, beta0),
                          (W1, b1, gamma1, beta1),
                          (W2, b2, gamma2, beta2)):
        m = _gather_max(h, binb, cnts)
        h = _dense(h, m, W, b, g, be)
    return h[:_N]


# speedup vs baseline: 5.2412x; 1.0245x over previous
import functools

import jax
import jax.numpy as jnp
from jax import lax
from jax.experimental import pallas as pl
from jax.experimental.pallas import tpu as pltpu
from jax.experimental.pallas import tpu_sc as plsc

_N = 10000
_D = 128
_E = 320000
_NC = 2
_NS = 16
_NW = _NC * _NS
_RPW = 320
_NPAD = _NW * _RPW
_PK = 16384
_CHUNK = 8000
_NCHUNKS = _E // _CHUNK
_FLUSH = 8192
_STG = _FLUSH + _CHUNK + 192
_CAP = _E + _STG
_G = 256

_mesh = plsc.VectorSubcoreMesh(core_axis_name="c", subcore_axis_name="s")


def _wid():
    return lax.axis_index("s") * _NC + lax.axis_index("c")


@functools.partial(
    pl.kernel,
    out_type=[
        jax.ShapeDtypeStruct((_NW * _CAP,), jnp.int32),
        jax.ShapeDtypeStruct((_NW * 16,), jnp.int32),
    ],
    mesh=_mesh,
    compiler_params=pltpu.CompilerParams(needs_layout_passes=False),
    scratch_types=[
        pltpu.VMEM((_CHUNK,), jnp.int32),
        pltpu.VMEM((_CHUNK,), jnp.int32),
        pltpu.VMEM((_STG,), jnp.int32),
        pltpu.VMEM((16,), jnp.int32),
        pltpu.SemaphoreType.DMA,
        pltpu.SemaphoreType.DMA,
    ],
)
def _bin_edges(pk_hbm, binb_hbm, cnt_hbm, c0, c1, stg, cvec, sem0, sem1):
    w = _wid()
    lo = w * _RPW
    hi = lo + _RPW

    def process(buf, carry):
        def group_body(i, off):
            pv = buf[pl.ds(i * 16, 16)]
            q = pv // _PK
            vd = pv - q * _PK
            m = (vd >= lo) & (vd < hi)
            csum = plsc.cumsum(m.astype(jnp.int32))
            plsc.store_scatter(stg, [csum + (off - 1)], pv, mask=m)
            return off + jnp.max(csum)

        off, written = carry
        off = lax.fori_loop(0, _CHUNK // 16, group_body, off)

        def do_flush(off, written):
            pltpu.sync_copy(
                stg.at[pl.ds(0, _FLUSH)],
                binb_hbm.at[pl.ds(pl.multiple_of(w * _CAP + written, 8), _FLUSH)])
            def shift_body(i, _):
                stg[pl.ds(i * 16, 16)] = stg[pl.ds(_FLUSH + i * 16, 16)]
                return 0
            lax.fori_loop(0, (_STG - _FLUSH) // 16, shift_body, 0)
            return off - _FLUSH, written + _FLUSH

        def no_flush(off, written):
            return off, written

        return lax.cond(off >= _FLUSH, do_flush, no_flush, off, written)

    pltpu.async_copy(pk_hbm.at[pl.ds(0, _CHUNK)], c0, sem0)

    def pair(p, carry):
        c = 2 * p
        pltpu.make_async_copy(pk_hbm.at[pl.ds(c * _CHUNK, _CHUNK)], c0, sem0).wait()
        pltpu.async_copy(pk_hbm.at[pl.ds((c + 1) * _CHUNK, _CHUNK)], c1, sem1)
        carry = process(c0, carry)
        pltpu.make_async_copy(pk_hbm.at[pl.ds((c + 1) * _CHUNK, _CHUNK)], c1, sem1).wait()

        @pl.when(c + 2 < _NCHUNKS)
        def _():
            pltpu.async_copy(pk_hbm.at[pl.ds((c + 2) * _CHUNK, _CHUNK)], c0, sem0)

        return process(c1, carry)

    off, written = lax.fori_loop(0, _NCHUNKS // 2, pair, (0, 0))
    pltpu.sync_copy(stg, binb_hbm.at[pl.ds(pl.multiple_of(w * _CAP + written, 8), _STG)])
    cvec[...] = jnp.full((16,), written + off, jnp.int32)
    pltpu.sync_copy(cvec, cnt_hbm.at[pl.ds(pl.multiple_of(w * 16, 8), 16)])


@functools.partial(
    pl.kernel,
    out_type=jax.ShapeDtypeStruct((_NPAD, _D), jnp.float32),
    mesh=_mesh,
    compiler_params=pltpu.CompilerParams(needs_layout_passes=False),
    scratch_types=[
        pltpu.VMEM((_RPW, _D), jnp.float32),
        pltpu.VMEM((_G,), jnp.int32),
        pltpu.VMEM((_G,), jnp.int32),
        pltpu.VMEM((_G,), jnp.int32),
        pltpu.VMEM((_G,), jnp.int32),
        pltpu.VMEM((_G + 16,), jnp.int32),
        pltpu.VMEM((_G + 16,), jnp.int32),
        pltpu.VMEM((_G, _D), jnp.float32),
        pltpu.VMEM((_G, _D), jnp.float32),
        pltpu.VMEM((16,), jnp.int32),
        pltpu.SemaphoreType.DMA,
        pltpu.SemaphoreType.DMA,
        pltpu.SemaphoreType.DMA,
        pltpu.SemaphoreType.DMA,
    ],
)
def _gather_max(h_hbm, binb_hbm, cnt_hbm, m_hbm,
                acc, pk0, pk1, sidx0, sidx1, dl0, dl1, rows0, rows1,
                cvec, sem0, sem1, psem0, psem1):
    w = _wid()
    lo = w * _RPW
    pltpu.sync_copy(h_hbm.at[pl.ds(pl.multiple_of(lo, 8), _RPW)], acc)
    pltpu.sync_copy(cnt_hbm.at[pl.ds(pl.multiple_of(w * 16, 8), 16)], cvec)
    cnt = jnp.max(cvec[...])
    ngroups = (cnt + _G - 1) // _G

    def pk_src(g):
        return binb_hbm.at[pl.ds(pl.multiple_of(w * _CAP + g * _G, 8), _G)]

    def unpack_issue(pk, sidx, dl, rows, sem):
        def unpack(i, _):
            pv = pk[pl.ds(i * 16, 16)]
            q = pv // _PK
            sidx[pl.ds(i * 16, 16)] = jnp.minimum(jnp.maximum(q, 0), _NPAD - 1)
            dl[pl.ds(i * 16, 16)] = pv - q * _PK - lo
            return 0

        lax.fori_loop(0, _G // 16, unpack, 0)
        pltpu.async_copy(h_hbm.at[sidx], rows, sem)

    def process(g, dl, rows):
        ne = jnp.maximum(0, jnp.minimum(_G, cnt - g * _G))

        def oct_body(p, _):
            e = p * 8
            v = dl[pl.ds(e, 16)]
            for k in range(8):
                d = v[k]
                for b in range(_D // 16):
                    sl = pl.ds(b * 16, 16)
                    acc[d, sl] = jnp.maximum(acc[d, sl], rows[e + k, sl])
            return 0

        lax.fori_loop(0, ne // 8, oct_body, 0)

        def tail_body(t, _):
            e = (ne // 8) * 8 + t
            d = dl[pl.ds(e, 16)][0]
            for b in range(_D // 16):
                sl = pl.ds(b * 16, 16)
                acc[d, sl] = jnp.maximum(acc[d, sl], rows[e, sl])
            return 0

        lax.fori_loop(0, ne % 8, tail_body, 0)

    @pl.when(ngroups > 0)
    def _():
        pltpu.async_copy(pk_src(0), pk0, psem0)
        pltpu.make_async_copy(pk_src(0), pk0, psem0).wait()
        unpack_issue(pk0, sidx0, dl0, rows0, sem0)

        @pl.when(1 < ngroups)
        def _():
            pltpu.async_copy(pk_src(1), pk1, psem1)

    def pair(p, _):
        g0 = 2 * p
        g1 = g0 + 1
        g2 = g0 + 2

        @pl.when(g1 < ngroups)
        def _():
            pltpu.make_async_copy(pk_src(g1), pk1, psem1).wait()

            @pl.when(g2 < ngroups)
            def _():
                pltpu.async_copy(pk_src(g2), pk0, psem0)

            unpack_issue(pk1, sidx1, dl1, rows1, sem1)

        pltpu.make_async_copy(h_hbm.at[sidx0], rows0, sem0).wait()
        process(g0, dl0, rows0)

        @pl.when(g1 < ngroups)
        def _():
            @pl.when(g2 < ngroups)
            def _():
                pltpu.make_async_copy(pk_src(g2), pk0, psem0).wait()

                @pl.when(g2 + 1 < ngroups)
                def _():
                    pltpu.async_copy(pk_src(g2 + 1), pk1, psem1)

                unpack_issue(pk0, sidx0, dl0, rows0, sem0)

            pltpu.make_async_copy(h_hbm.at[sidx1], rows1, sem1).wait()
            process(g1, dl1, rows1)

        return 0

    lax.fori_loop(0, (ngroups + 1) // 2, pair, 0)
    pltpu.sync_copy(acc, m_hbm.at[pl.ds(pl.multiple_of(lo, 8), _RPW)])


def _dense_body(h_ref, m_ref, w_ref, b_ref, g_ref, be_ref, o_ref):
    hh = h_ref[...]
    rel = m_ref[...] - hh
    z = jnp.dot(hh, w_ref[:_D, :], preferred_element_type=jnp.float32)
    z = z + jnp.dot(rel, w_ref[_D:, :], preferred_element_type=jnp.float32)
    z = z + b_ref[...]
    mu = jnp.mean(z, axis=-1, keepdims=True)
    var = jnp.mean((z - mu) ** 2, axis=-1, keepdims=True)
    y = (z - mu) * lax.rsqrt(var + 1e-5) * g_ref[...] + be_ref[...]
    o_ref[...] = jnp.where(y > 0, y, jnp.exp(jnp.minimum(y, 0.0)) - 1.0)


_BLK = 512


def _dense(h, m, W, b, gamma, beta):
    return pl.pallas_call(
        _dense_body,
        grid=(_NPAD // _BLK,),
        in_specs=[
            pl.BlockSpec((_BLK, _D), lambda i: (i, 0)),
            pl.BlockSpec((_BLK, _D), lambda i: (i, 0)),
            pl.BlockSpec((2 * _D, _D), lambda i: (0, 0)),
            pl.BlockSpec((1, _D), lambda i: (0, 0)),
            pl.BlockSpec((1, _D), lambda i: (0, 0)),
            pl.BlockSpec((1, _D), lambda i: (0, 0)),
        ],
        out_specs=pl.BlockSpec((_BLK, _D), lambda i: (i, 0)),
        out_shape=jax.ShapeDtypeStruct((_NPAD, _D), jnp.float32),
    )(h, m, W, b[None], gamma[None], beta[None])


def kernel(features, edge_index, W0, b0, gamma0, beta0,
           W1, b1, gamma1, beta1, W2, b2, gamma2, beta2):
    src = edge_index[0].astype(jnp.int32)
    dst = edge_index[1].astype(jnp.int32)
    pk = src * _PK + dst
    h = jnp.pad(features, ((0, _NPAD - _N), (0, 0)))
    binb, cnts = _bin_edges(pk)
    for (W, b, g, be) in ((W0, b0, gamma0, beta0),
                          (W1, b1, gamma1, beta1),
                          (W2, b2, gamma2, beta2)):
        m = _gather_max(h, binb, cnts)
        h = _dense(h, m, W, b, g, be)
    return h[:_N]
